# baseline (speedup 1.0000x reference)
import jax
import jax.numpy as jnp
from jax import lax
from jax.experimental import pallas as pl
from jax.experimental.pallas import tpu as pltpu
from jax.experimental.pallas import tpu_sc as plsc

_B, _C, _K, _L = 16, 144, 128, 256
_C_TIME = 128
_NC, _NS = 2, 16
_NLANE = 16


def _tabrep_body(tab_t_ref, rep_ref):
    tab = tab_t_ref[...]
    rep_ref[...] = jnp.broadcast_to(tab[:, :, None], (_C - _C_TIME, _K, _NLANE))


def _tabrep(tab_t):
    return pl.pallas_call(
        _tabrep_body,
        in_specs=[pl.BlockSpec((_C - _C_TIME, _K), lambda: (0, 0))],
        out_specs=pl.BlockSpec((_C - _C_TIME, _K, _NLANE), lambda: (0, 0, 0)),
        out_shape=jax.ShapeDtypeStruct((_C - _C_TIME, _K, _NLANE), jnp.float32),
    )(tab_t)


def _sc_body(rep_hbm, out_hbm, rep_v, pat_v, sem):
    wid = lax.axis_index("s") * _NC + lax.axis_index("c")
    j = wid % 16
    bhalf = wid // 16
    pltpu.sync_copy(rep_hbm.at[j], rep_v)
    for k in range(_K):
        vec = rep_v[k]
        for i in range(_L // _NLANE):
            pat_v[k, pl.ds(i * _NLANE, _NLANE)] = vec
    copies = []
    for i in range(8):
        b = bhalf * 8 + i
        copies.append(pltpu.async_copy(pat_v, out_hbm.at[b, _C_TIME + j], sem))
    for c in copies:
        c.wait()


def _sc_fill(tab_rep):
    mesh = plsc.VectorSubcoreMesh(
        core_axis_name="c", subcore_axis_name="s",
        num_cores=_NC, num_subcores=_NS,
    )
    f = pl.kernel(
        _sc_body,
        out_type=jax.ShapeDtypeStruct((_B, _C, _K, _L), jnp.float32),
        mesh=mesh,
        scratch_types=[
            pltpu.VMEM((_K, _NLANE), jnp.float32),
            pltpu.VMEM((_K, _L), jnp.float32),
            pltpu.SemaphoreType.DMA,
        ],
    )
    return f(tab_rep)


def _time_body(in_ref, out_ref, scratch, sems):
    del in_ref
    ci = jax.lax.broadcasted_iota(jnp.int32, (_C_TIME, _L), 0)
    li = jax.lax.broadcasted_iota(jnp.int32, (_C_TIME, _L), 1)
    c_rem = ci - (ci // 2) * 2
    c_even = (ci - c_rem).astype(jnp.float32)
    ln10000 = 9.210340371976184
    div = jnp.exp(c_even * (-ln10000 / 128.0))
    angle = li.astype(jnp.float32) * div
    pe = jnp.where(c_rem == 0, jnp.sin(angle), jnp.cos(angle))
    scratch[...] = jnp.broadcast_to(pe[:, None, :], (_C_TIME, _K, _L))
    for b in range(_B):
        pltpu.make_async_copy(
            scratch, out_ref.at[b, pl.ds(0, _C_TIME)], sems.at[b]
        ).start()
    for b in range(_B):
        pltpu.make_async_copy(
            scratch, out_ref.at[b, pl.ds(0, _C_TIME)], sems.at[b]
        ).wait()


def _time_fill(big):
    return pl.pallas_call(
        _time_body,
        in_specs=[pl.BlockSpec(memory_space=pl.ANY)],
        out_specs=pl.BlockSpec(memory_space=pl.ANY),
        out_shape=jax.ShapeDtypeStruct((_B, _C, _K, _L), jnp.float32),
        scratch_shapes=[
            pltpu.VMEM((_C_TIME, _K, _L), jnp.float32),
            pltpu.SemaphoreType.DMA((_B,)),
        ],
        input_output_aliases={0: 0},
    )(big)


def kernel(cond_mask, table):
    del cond_mask
    tab_t = table.T
    tab_rep = _tabrep(tab_t)
    big = _sc_fill(tab_rep)
    return _time_fill(big)


# speedup vs baseline: 1.0372x; 1.0372x over previous
import jax
import jax.numpy as jnp
from jax import lax
from jax.experimental import pallas as pl
from jax.experimental.pallas import tpu as pltpu
from jax.experimental.pallas import tpu_sc as plsc

_B, _C, _K, _L = 16, 144, 128, 256
_C_TIME = 128
_NC, _NS = 2, 16
_NCHUNK = 8
_CCH = _C_TIME // _NCHUNK


def _patterns_body(tab_ref, pat_ref):
    tab_t = tab_ref[...].T
    pat_ref[...] = jnp.broadcast_to(tab_t[:, :, None], (_C - _C_TIME, _K, _L))


def _patterns(table):
    return pl.pallas_call(
        _patterns_body,
        in_specs=[pl.BlockSpec((_K, _C - _C_TIME), lambda: (0, 0))],
        out_specs=pl.BlockSpec((_C - _C_TIME, _K, _L), lambda: (0, 0, 0)),
        out_shape=jax.ShapeDtypeStruct((_C - _C_TIME, _K, _L), jnp.float32),
    )(table)


def _sc_body(pat_hbm, out_hbm, pat_v, sem):
    wid = lax.axis_index("s") * _NC + lax.axis_index("c")
    j = wid % 16
    bhalf = wid // 16
    pltpu.sync_copy(pat_hbm.at[j], pat_v)
    copies = []
    for i in range(8):
        b = bhalf * 8 + i
        copies.append(pltpu.async_copy(pat_v, out_hbm.at[b, _C_TIME + j], sem))
    for c in copies:
        c.wait()


def _sc_fill(patterns):
    mesh = plsc.VectorSubcoreMesh(
        core_axis_name="c", subcore_axis_name="s",
        num_cores=_NC, num_subcores=_NS,
    )
    f = pl.kernel(
        _sc_body,
        out_type=jax.ShapeDtypeStruct((_B, _C, _K, _L), jnp.float32),
        mesh=mesh,
        scratch_types=[
            pltpu.VMEM((_K, _L), jnp.float32),
            pltpu.SemaphoreType.DMA,
        ],
    )
    return f(patterns)


def _time_body(in_ref, out_ref, scratch, sems):
    del in_ref
    ci = jax.lax.broadcasted_iota(jnp.int32, (_C_TIME, _L), 0)
    li = jax.lax.broadcasted_iota(jnp.int32, (_C_TIME, _L), 1)
    c_rem = ci - (ci // 2) * 2
    c_even = (ci - c_rem).astype(jnp.float32)
    ln10000 = 9.210340371976184
    div = jnp.exp(c_even * (-ln10000 / 128.0))
    angle = li.astype(jnp.float32) * div
    pe = jnp.where(c_rem == 0, jnp.sin(angle), jnp.cos(angle))
    copies = []
    for g in range(_NCHUNK):
        c0 = g * _CCH
        scratch[pl.ds(c0, _CCH), :, :] = jnp.broadcast_to(
            pe[c0:c0 + _CCH, None, :], (_CCH, _K, _L)
        )
        for b in range(_B):
            copies.append(
                pltpu.make_async_copy(
                    scratch.at[pl.ds(c0, _CCH)],
                    out_ref.at[b, pl.ds(c0, _CCH)],
                    sems.at[g, b],
                )
            )
            copies[-1].start()
    for c in copies:
        c.wait()


def _time_fill(big):
    return pl.pallas_call(
        _time_body,
        in_specs=[pl.BlockSpec(memory_space=pl.ANY)],
        out_specs=pl.BlockSpec(memory_space=pl.ANY),
        out_shape=jax.ShapeDtypeStruct((_B, _C, _K, _L), jnp.float32),
        scratch_shapes=[
            pltpu.VMEM((_C_TIME, _K, _L), jnp.float32),
            pltpu.SemaphoreType.DMA((_NCHUNK, _B)),
        ],
        input_output_aliases={0: 0},
    )(big)


def kernel(cond_mask, table):
    del cond_mask
    patterns = _patterns(table)
    big = _sc_fill(patterns)
    return _time_fill(big)
